# Initial kernel scaffold; baseline (speedup 1.0000x reference)
import jax
import jax.numpy as jnp
from jax.experimental import pallas as pl


def kernel(x, index, anchors):
    raise NotImplementedError("write your pallas kernel here")


# speedup vs baseline: 5.8293x; 5.8293x over previous
import jax
import jax.numpy as jnp
from jax import lax
from jax.experimental import pallas as pl

T = 50
D = 50
BS = 4
RI = 12
RC = 16
NA = T * D


def _body(x_ref, st_ref, en_ref, out_ref):
    xb = x_ref[0]
    st = st_ref[...]
    en = en_ref[...]

    ip = lax.dot_general(xb, xb, (((0,), (0,)), ((), ())),
                         preferred_element_type=jnp.float32)
    xx = jnp.sum(xb * xb * xb, axis=0, keepdims=True)
    m = jnp.sum(xb, axis=0, keepdims=True) * (1.0 / T)
    score = 2.0 * ip - xx

    jio = lax.broadcasted_iota(jnp.int32, (T, T), 1)
    mb = jnp.broadcast_to(m, (T, T))
    feats = []
    work = score
    for _ in range(3):
        rmax = jnp.max(work, axis=1, keepdims=True)
        cand = jnp.where(work == rmax, jio, T)
        idx = jnp.min(cand, axis=1, keepdims=True)
        onehot = jio == idx
        feats.append(jnp.sum(jnp.where(onehot, mb, 0.0), axis=1, keepdims=True))
        work = jnp.where(onehot, -jnp.inf, work)
    f0, f1, f2 = feats

    ln = jnp.maximum(en - st, 1.0)

    binsz = ln * (1.0 / RI)
    tio = lax.broadcasted_iota(jnp.int32, (T, NA), 0)
    for r in range(RI):
        pos = st + binsz * (r + 0.5)
        valid = (pos >= -1.0) & (pos <= float(T))
        pos_c = jnp.clip(pos, 0.0, float(T - 1))
        lo = jnp.floor(pos_c)
        hi = jnp.minimum(lo + 1.0, float(T - 1))
        w = pos_c - lo
        lo_i = lo.astype(jnp.int32)
        hi_i = hi.astype(jnp.int32)
        wq = jnp.where(tio == lo_i, 1.0 - w, 0.0) + jnp.where(tio == hi_i, w, 0.0)
        wq = jnp.where(valid, wq, 0.0)
        out_ref[0, :, r, :] = lax.dot_general(
            xb, wq, (((1,), (0,)), ((), ())),
            preferred_element_type=jnp.float32)

    binsz = ln * (1.0 / RC)
    for r in range(RC):
        pos = st + binsz * (r + 0.5)
        valid = (pos >= -1.0) & (pos <= 3.0)
        pos_c = jnp.clip(pos, 0.0, 2.0)
        lo = jnp.floor(pos_c)
        hi = jnp.minimum(lo + 1.0, 2.0)
        w = pos_c - lo
        a0 = jnp.where(lo == 0.0, 1.0 - w, 0.0) + jnp.where(hi == 0.0, w, 0.0)
        a1 = jnp.where(lo == 1.0, 1.0 - w, 0.0) + jnp.where(hi == 1.0, w, 0.0)
        a2 = jnp.where(lo == 2.0, 1.0 - w, 0.0) + jnp.where(hi == 2.0, w, 0.0)
        a0 = jnp.where(valid, a0, 0.0)
        a1 = jnp.where(valid, a1, 0.0)
        a2 = jnp.where(valid, a2, 0.0)
        out_ref[0, :, RI + r, :] = f0 * a0 + f1 * a1 + f2 * a2


def kernel(x, index, anchors):
    del index
    st = anchors[:NA, 1].reshape(T, D).T.reshape(1, NA)
    en = anchors[:NA, 2].reshape(T, D).T.reshape(1, NA)

    out = pl.pallas_call(
        _body,
        grid=(BS,),
        in_specs=[
            pl.BlockSpec((1, T, T), lambda b: (b, 0, 0)),
            pl.BlockSpec((1, NA), lambda b: (0, 0)),
            pl.BlockSpec((1, NA), lambda b: (0, 0)),
        ],
        out_specs=pl.BlockSpec((1, T, RI + RC, NA), lambda b: (b, 0, 0, 0)),
        out_shape=jax.ShapeDtypeStruct((BS, T, RI + RC, NA), jnp.float32),
    )(x, st, en)
    return out.reshape(BS, T * (RI + RC), D, T)
